# Initial kernel scaffold; baseline (speedup 1.0000x reference)
import jax
import jax.numpy as jnp
from jax.experimental import pallas as pl


def kernel(features_1, edge_index_1, features_2, edge_index_2, W1, b1, W2, b2, W3, b3, att_W, tn_W, tn_Wb, tn_bias, fc1_W, fc1_b, sc_W, sc_b):
    raise NotImplementedError("write your pallas kernel here")


# speedup vs baseline: 13.8449x; 13.8449x over previous
import functools

import jax
import jax.numpy as jnp
from jax import lax
from jax.experimental import pallas as pl
from jax.experimental.pallas import tpu as pltpu
from jax.experimental.pallas import tpu_sc as plsc

N = 10000
E = 320000
D = 128
F1, F2, F3 = 128, 64, 32
T = 16
BN = 16

NPAD = 10240
NTILES = 16
RPT = NPAD // NTILES
CHUNK = 128
IB = 16
NBLK = 10
NCHUNK = IB * NBLK
EPT = NCHUNK * CHUNK
E_PAD = NTILES * EPT
FW = 128


def _sc_mesh():
    return plsc.VectorSubcoreMesh(core_axis_name="c", subcore_axis_name="s",
                                  num_cores=2, num_subcores=NTILES)


@functools.lru_cache(maxsize=None)
def _make_agg():
    @functools.partial(
        pl.kernel,
        out_type=jax.ShapeDtypeStruct((2 * NPAD, FW), jnp.float32),
        mesh=_sc_mesh(),
        scratch_types=[
            pltpu.VMEM_SHARED((NPAD, FW), jnp.float32),
            pltpu.VMEM((IB, CHUNK), jnp.int32),
            pltpu.VMEM((IB, CHUNK), jnp.int32),
            pltpu.VMEM((CHUNK, FW), jnp.float32),
            pltpu.SemaphoreType.DMA,
        ],
    )
    def agg(g_hbm, src_hbm, dst_hbm, out_hbm, acc_sh, sidx, didx, rows, sem):
        c = lax.axis_index("c")
        s = lax.axis_index("s")
        w = c * NTILES + s
        base = c * NPAD + s * RPT
        pltpu.sync_copy(g_hbm.at[pl.ds(base, RPT)], acc_sh.at[pl.ds(s * RPT, RPT)])
        plsc.subcore_barrier()

        def blk(bi, carry):
            ib = w * NCHUNK + bi * IB
            pltpu.sync_copy(src_hbm.at[pl.ds(ib, IB)], sidx)
            pltpu.sync_copy(dst_hbm.at[pl.ds(ib, IB)], didx)

            def body(k, carry2):
                pltpu.async_copy(g_hbm.at[sidx.at[k]], rows, sem).wait()
                pltpu.sync_copy(rows, acc_sh.at[didx.at[k]], add=True)
                return carry2

            lax.fori_loop(0, IB, body, 0)
            return carry

        lax.fori_loop(0, NBLK, blk, 0)
        plsc.subcore_barrier()
        pltpu.sync_copy(acc_sh.at[pl.ds(s * RPT, RPT)], out_hbm.at[pl.ds(base, RPT)])

    return agg


@functools.lru_cache(maxsize=None)
def _make_degree():
    @functools.partial(
        pl.kernel,
        out_type=jax.ShapeDtypeStruct((2 * NPAD, FW), jnp.float32),
        mesh=_sc_mesh(),
        scratch_types=[
            pltpu.VMEM_SHARED((NPAD, FW), jnp.float32),
            pltpu.VMEM((IB, CHUNK), jnp.int32),
            pltpu.VMEM((CHUNK, FW), jnp.float32),
            pltpu.SemaphoreType.DMA,
        ],
    )
    def degree(zeros_hbm, ones_hbm, dst_hbm, out_hbm, acc_sh, didx, ones_v, sem):
        c = lax.axis_index("c")
        s = lax.axis_index("s")
        w = c * NTILES + s
        base = c * NPAD + s * RPT
        pltpu.sync_copy(zeros_hbm.at[pl.ds(s * RPT, RPT)],
                        acc_sh.at[pl.ds(s * RPT, RPT)])
        pltpu.sync_copy(ones_hbm, ones_v)
        plsc.subcore_barrier()

        def blk(bi, carry):
            pltpu.sync_copy(dst_hbm.at[pl.ds(w * NCHUNK + bi * IB, IB)], didx)

            def body(k, carry2):
                pltpu.sync_copy(ones_v, acc_sh.at[didx.at[k]], add=True)
                return carry2

            lax.fori_loop(0, IB, body, 0)
            return carry

        lax.fori_loop(0, NBLK, blk, 0)
        plsc.subcore_barrier()
        pltpu.sync_copy(acc_sh.at[pl.ds(s * RPT, RPT)], out_hbm.at[pl.ds(base, RPT)])

    return degree


_BM = 1024


def _dense_first(xcat, deg, W):
    M, K = xcat.shape
    F = W.shape[1]

    def body(x_ref, deg_ref, w_ref, o_ref):
        dinv = lax.rsqrt(deg_ref[...][:, 0:1] + 1.0)
        o_ref[...] = dinv * jnp.dot(x_ref[...], w_ref[...],
                                    preferred_element_type=jnp.float32)

    return pl.pallas_call(
        body,
        grid=(M // _BM,),
        in_specs=[
            pl.BlockSpec((_BM, K), lambda i: (i, 0)),
            pl.BlockSpec((_BM, FW), lambda i: (i, 0)),
            pl.BlockSpec((K, F), lambda i: (0, 0)),
        ],
        out_specs=pl.BlockSpec((_BM, F), lambda i: (i, 0)),
        out_shape=jax.ShapeDtypeStruct((M, F), jnp.float32),
    )(xcat, deg, W)


def _dense_mid(acc, deg, b_row, W):
    M, K = acc.shape
    F = W.shape[1]

    def body(a_ref, deg_ref, b_ref, w_ref, o_ref):
        dinv = lax.rsqrt(deg_ref[...][:, 0:1] + 1.0)
        h = jnp.maximum(dinv * a_ref[...] + b_ref[...], 0.0)
        o_ref[...] = dinv * jnp.dot(h, w_ref[...],
                                    preferred_element_type=jnp.float32)

    return pl.pallas_call(
        body,
        grid=(M // _BM,),
        in_specs=[
            pl.BlockSpec((_BM, K), lambda i: (i, 0)),
            pl.BlockSpec((_BM, FW), lambda i: (i, 0)),
            pl.BlockSpec((1, K), lambda i: (0, 0)),
            pl.BlockSpec((K, F), lambda i: (0, 0)),
        ],
        out_specs=pl.BlockSpec((_BM, F), lambda i: (i, 0)),
        out_shape=jax.ShapeDtypeStruct((M, F), jnp.float32),
    )(acc, deg, b_row, W)


def _final_body(acc_ref, deg_ref, b3_ref, attw_ref, tnw_ref, tnwbt_ref,
                tnb_ref, fc1wt_ref, fc1b_ref, scwt_ref, scb_ref,
                score_ref, p1_ref, p2_ref):
    b3 = b3_ref[...]
    attw = attw_ref[...]

    def pooled_rows(lo):
        a = acc_ref[pl.ds(lo, N), 0:F3]
        d = deg_ref[pl.ds(lo, N), 0:1]
        dinv = lax.rsqrt(d + 1.0)
        emb = dinv * a + b3
        mean = jnp.mean(emb, axis=0, keepdims=True)
        ctx = jnp.tanh(jnp.dot(mean, attw,
                               preferred_element_type=jnp.float32))
        logits = lax.dot_general(emb, ctx, (((1,), (1,)), ((), ())),
                                 preferred_element_type=jnp.float32)
        sig = jax.nn.sigmoid(logits)
        pooled_col = lax.dot_general(emb, sig, (((0,), (0,)), ((), ())),
                                     preferred_element_type=jnp.float32)
        pooled_row = lax.dot_general(sig, emb, (((0,), (0,)), ((), ())),
                                     preferred_element_type=jnp.float32)
        return pooled_col, pooled_row

    p1c, e1r = pooled_rows(0)
    p2c, e2r = pooled_rows(NPAD)
    p1_ref[...] = p1c
    p2_ref[...] = p2c

    y = jnp.dot(e2r, tnw_ref[...], preferred_element_type=jnp.float32)
    e1_tiled = jnp.concatenate([e1r] * T, axis=1)
    z = y * e1_tiled
    rr = lax.broadcasted_iota(jnp.int32, (T * F3, T), 0)
    cc = lax.broadcasted_iota(jnp.int32, (T * F3, T), 1)
    sel = (rr // F3 == cc).astype(jnp.float32)
    scoring = jnp.dot(z, sel, preferred_element_type=jnp.float32)

    comb = jnp.concatenate([e1r, e2r], axis=1)
    block = jnp.dot(comb, tnwbt_ref[...], preferred_element_type=jnp.float32)
    s = jnp.maximum(scoring + block + tnb_ref[...], 0.0)
    s2 = jnp.maximum(jnp.dot(s, fc1wt_ref[...],
                             preferred_element_type=jnp.float32) + fc1b_ref[...], 0.0)
    score_ref[...] = jax.nn.sigmoid(
        jnp.dot(s2, scwt_ref[...], preferred_element_type=jnp.float32) + scb_ref[...])


def _final(acc3, deg, b3_row, att_W, tn_wcols, tn_wbt, tn_b_row,
           fc1_wt, fc1_b_row, sc_wt, sc_b_row):
    return pl.pallas_call(
        _final_body,
        out_shape=(
            jax.ShapeDtypeStruct((1, 1), jnp.float32),
            jax.ShapeDtypeStruct((F3, 1), jnp.float32),
            jax.ShapeDtypeStruct((F3, 1), jnp.float32),
        ),
    )(acc3, deg, b3_row, att_W, tn_wcols, tn_wbt, tn_b_row,
      fc1_wt, fc1_b_row, sc_wt, sc_b_row)


def kernel(features_1, edge_index_1, features_2, edge_index_2,
           W1, b1, W2, b2, W3, b3, att_W, tn_W, tn_Wb, tn_bias,
           fc1_W, fc1_b, sc_W, sc_b):
    f32 = jnp.float32
    src1, dst1 = edge_index_1[0], edge_index_1[1]
    src2, dst2 = edge_index_2[0], edge_index_2[1]

    padn = E_PAD - E
    padidx = (N + (jnp.arange(padn, dtype=jnp.int32) % 16)).astype(jnp.int32)

    def pad_edges(a):
        return jnp.concatenate([a.astype(jnp.int32), padidx])

    src_glob = jnp.concatenate(
        [pad_edges(src1), pad_edges(src2) + NPAD]).reshape(2 * NTILES * NCHUNK, CHUNK)
    dst_all = jnp.concatenate(
        [pad_edges(dst1), pad_edges(dst2)]).reshape(2 * NTILES * NCHUNK, CHUNK)

    zeros_w = jnp.zeros((NPAD, FW), f32)
    ones_chunk = jnp.ones((CHUNK, FW), f32)
    deg = _make_degree()(zeros_w, ones_chunk, dst_all)

    zrows = jnp.zeros((NPAD - N, D), f32)
    xcat = jnp.concatenate([features_1, zrows, features_2, zrows])

    W2p = jnp.zeros((F1, FW), f32).at[:, :F2].set(W2)
    W3p = jnp.zeros((FW, FW), f32).at[:F2, :F3].set(W3)
    b2p = jnp.zeros((FW,), f32).at[:F2].set(b2)

    agg = _make_agg()
    g1 = _dense_first(xcat, deg, W1)
    acc1 = agg(g1, src_glob, dst_all)
    g2 = _dense_mid(acc1, deg, b1.reshape(1, -1), W2p)
    acc2 = agg(g2, src_glob, dst_all)
    g3 = _dense_mid(acc2, deg, b2p.reshape(1, -1), W3p)
    acc3 = agg(g3, src_glob, dst_all)

    tn_wcols = jnp.transpose(tn_W, (1, 2, 0)).reshape(F3, T * F3)
    score, p1, p2 = _final(
        acc3, deg, b3.reshape(1, -1), att_W, tn_wcols,
        tn_Wb.T, tn_bias.reshape(1, -1), fc1_W.T, fc1_b.reshape(1, -1),
        sc_W.T, sc_b.reshape(1, -1))
    return (score, p1, p2)


# speedup vs baseline: 16.0827x; 1.1616x over previous
import functools

import jax
import jax.numpy as jnp
from jax import lax
from jax.experimental import pallas as pl
from jax.experimental.pallas import tpu as pltpu
from jax.experimental.pallas import tpu_sc as plsc

N = 10000
E = 320000
D = 128
F1, F2, F3 = 128, 64, 32
T = 16
BN = 16

NPAD = 10240
NTILES = 16
RPT = NPAD // NTILES
CHUNK = 128
IB = 16
NBLK = 10
NCHUNK = IB * NBLK
EPT = NCHUNK * CHUNK
E_PAD = NTILES * EPT
FW = 128


def _sc_mesh():
    return plsc.VectorSubcoreMesh(core_axis_name="c", subcore_axis_name="s",
                                  num_cores=2, num_subcores=NTILES)


@functools.lru_cache(maxsize=None)
def _make_agg():
    @functools.partial(
        pl.kernel,
        out_type=jax.ShapeDtypeStruct((2 * NPAD, FW), jnp.float32),
        mesh=_sc_mesh(),
        scratch_types=[
            pltpu.VMEM_SHARED((NPAD, FW), jnp.float32),
            pltpu.VMEM((IB, CHUNK), jnp.int32),
            pltpu.VMEM((IB, CHUNK), jnp.int32),
            pltpu.VMEM((CHUNK, FW), jnp.float32),
            pltpu.VMEM((CHUNK, FW), jnp.float32),
            pltpu.SemaphoreType.DMA,
            pltpu.SemaphoreType.DMA,
            pltpu.SemaphoreType.DMA,
            pltpu.SemaphoreType.DMA,
        ],
    )
    def agg(g_hbm, src_hbm, dst_hbm, out_hbm, acc_sh, sidx, didx,
            rows0, rows1, gsem0, gsem1, ssem0, ssem1):
        c = lax.axis_index("c")
        s = lax.axis_index("s")
        w = c * NTILES + s
        base = c * NPAD + s * RPT
        pltpu.sync_copy(g_hbm.at[pl.ds(base, RPT)], acc_sh.at[pl.ds(s * RPT, RPT)])
        plsc.subcore_barrier()

        def drain_scatter(sem, buf):
            pltpu.make_async_copy(buf, acc_sh.at[pl.ds(0, CHUNK)], sem).wait()

        def drain_gather(sem, buf):
            pltpu.make_async_copy(g_hbm.at[pl.ds(0, CHUNK)], buf, sem).wait()

        def blk(bi, carry):
            @pl.when(bi > 0)
            def _():
                drain_scatter(ssem0, rows0)
                drain_scatter(ssem1, rows1)

            ib = w * NCHUNK + bi * IB
            pltpu.sync_copy(src_hbm.at[pl.ds(ib, IB)], sidx)
            pltpu.sync_copy(dst_hbm.at[pl.ds(ib, IB)], didx)

            def pair(m, carry2):
                ka = 2 * m
                kb = 2 * m + 1

                @pl.when(m > 0)
                def _():
                    drain_scatter(ssem0, rows0)

                pltpu.async_copy(g_hbm.at[sidx.at[ka]], rows0, gsem0)

                @pl.when(m > 0)
                def _():
                    drain_scatter(ssem1, rows1)

                pltpu.async_copy(g_hbm.at[sidx.at[kb]], rows1, gsem1)
                drain_gather(gsem0, rows0)
                pltpu.async_copy(rows0, acc_sh.at[didx.at[ka]], ssem0, add=True)
                drain_gather(gsem1, rows1)
                pltpu.async_copy(rows1, acc_sh.at[didx.at[kb]], ssem1, add=True)
                return carry2

            lax.fori_loop(0, IB // 2, pair, 0)
            return carry

        lax.fori_loop(0, NBLK, blk, 0)
        drain_scatter(ssem0, rows0)
        drain_scatter(ssem1, rows1)
        plsc.subcore_barrier()
        pltpu.sync_copy(acc_sh.at[pl.ds(s * RPT, RPT)], out_hbm.at[pl.ds(base, RPT)])

    return agg


@functools.lru_cache(maxsize=None)
def _make_degree():
    @functools.partial(
        pl.kernel,
        out_type=jax.ShapeDtypeStruct((2 * NPAD, FW), jnp.float32),
        mesh=_sc_mesh(),
        scratch_types=[
            pltpu.VMEM_SHARED((NPAD, FW), jnp.float32),
            pltpu.VMEM((IB, CHUNK), jnp.int32),
            pltpu.VMEM((CHUNK, FW), jnp.float32),
            pltpu.SemaphoreType.DMA,
        ],
    )
    def degree(zeros_hbm, ones_hbm, dst_hbm, out_hbm, acc_sh, didx, ones_v, sem):
        c = lax.axis_index("c")
        s = lax.axis_index("s")
        w = c * NTILES + s
        base = c * NPAD + s * RPT
        pltpu.sync_copy(zeros_hbm.at[pl.ds(s * RPT, RPT)],
                        acc_sh.at[pl.ds(s * RPT, RPT)])
        pltpu.sync_copy(ones_hbm, ones_v)
        plsc.subcore_barrier()

        def blk(bi, carry):
            pltpu.sync_copy(dst_hbm.at[pl.ds(w * NCHUNK + bi * IB, IB)], didx)

            def body(k, carry2):
                pltpu.sync_copy(ones_v, acc_sh.at[didx.at[k]], add=True)
                return carry2

            lax.fori_loop(0, IB, body, 0)
            return carry

        lax.fori_loop(0, NBLK, blk, 0)
        plsc.subcore_barrier()
        pltpu.sync_copy(acc_sh.at[pl.ds(s * RPT, RPT)], out_hbm.at[pl.ds(base, RPT)])

    return degree


_BM = 1024


def _dense_first(xcat, deg, W):
    M, K = xcat.shape
    F = W.shape[1]

    def body(x_ref, deg_ref, w_ref, o_ref):
        dinv = lax.rsqrt(deg_ref[...][:, 0:1] + 1.0)
        o_ref[...] = dinv * jnp.dot(x_ref[...], w_ref[...],
                                    preferred_element_type=jnp.float32)

    return pl.pallas_call(
        body,
        grid=(M // _BM,),
        in_specs=[
            pl.BlockSpec((_BM, K), lambda i: (i, 0)),
            pl.BlockSpec((_BM, FW), lambda i: (i, 0)),
            pl.BlockSpec((K, F), lambda i: (0, 0)),
        ],
        out_specs=pl.BlockSpec((_BM, F), lambda i: (i, 0)),
        out_shape=jax.ShapeDtypeStruct((M, F), jnp.float32),
    )(xcat, deg, W)


def _dense_mid(acc, deg, b_row, W):
    M, K = acc.shape
    F = W.shape[1]

    def body(a_ref, deg_ref, b_ref, w_ref, o_ref):
        dinv = lax.rsqrt(deg_ref[...][:, 0:1] + 1.0)
        h = jnp.maximum(dinv * a_ref[...] + b_ref[...], 0.0)
        o_ref[...] = dinv * jnp.dot(h, w_ref[...],
                                    preferred_element_type=jnp.float32)

    return pl.pallas_call(
        body,
        grid=(M // _BM,),
        in_specs=[
            pl.BlockSpec((_BM, K), lambda i: (i, 0)),
            pl.BlockSpec((_BM, FW), lambda i: (i, 0)),
            pl.BlockSpec((1, K), lambda i: (0, 0)),
            pl.BlockSpec((K, F), lambda i: (0, 0)),
        ],
        out_specs=pl.BlockSpec((_BM, F), lambda i: (i, 0)),
        out_shape=jax.ShapeDtypeStruct((M, F), jnp.float32),
    )(acc, deg, b_row, W)


def _final_body(acc_ref, deg_ref, b3_ref, attw_ref, tnw_ref, tnwbt_ref,
                tnb_ref, fc1wt_ref, fc1b_ref, scwt_ref, scb_ref,
                score_ref, p1_ref, p2_ref):
    b3 = b3_ref[...]
    attw = attw_ref[...]

    def pooled_rows(lo):
        a = acc_ref[pl.ds(lo, N), 0:F3]
        d = deg_ref[pl.ds(lo, N), 0:1]
        dinv = lax.rsqrt(d + 1.0)
        emb = dinv * a + b3
        mean = jnp.mean(emb, axis=0, keepdims=True)
        ctx = jnp.tanh(jnp.dot(mean, attw,
                               preferred_element_type=jnp.float32))
        logits = lax.dot_general(emb, ctx, (((1,), (1,)), ((), ())),
                                 preferred_element_type=jnp.float32)
        sig = jax.nn.sigmoid(logits)
        pooled_col = lax.dot_general(emb, sig, (((0,), (0,)), ((), ())),
                                     preferred_element_type=jnp.float32)
        pooled_row = lax.dot_general(sig, emb, (((0,), (0,)), ((), ())),
                                     preferred_element_type=jnp.float32)
        return pooled_col, pooled_row

    p1c, e1r = pooled_rows(0)
    p2c, e2r = pooled_rows(NPAD)
    p1_ref[...] = p1c
    p2_ref[...] = p2c

    y = jnp.dot(e2r, tnw_ref[...], preferred_element_type=jnp.float32)
    e1_tiled = jnp.concatenate([e1r] * T, axis=1)
    z = y * e1_tiled
    rr = lax.broadcasted_iota(jnp.int32, (T * F3, T), 0)
    cc = lax.broadcasted_iota(jnp.int32, (T * F3, T), 1)
    sel = (rr // F3 == cc).astype(jnp.float32)
    scoring = jnp.dot(z, sel, preferred_element_type=jnp.float32)

    comb = jnp.concatenate([e1r, e2r], axis=1)
    block = jnp.dot(comb, tnwbt_ref[...], preferred_element_type=jnp.float32)
    s = jnp.maximum(scoring + block + tnb_ref[...], 0.0)
    s2 = jnp.maximum(jnp.dot(s, fc1wt_ref[...],
                             preferred_element_type=jnp.float32) + fc1b_ref[...], 0.0)
    score_ref[...] = jax.nn.sigmoid(
        jnp.dot(s2, scwt_ref[...], preferred_element_type=jnp.float32) + scb_ref[...])


def _final(acc3, deg, b3_row, att_W, tn_wcols, tn_wbt, tn_b_row,
           fc1_wt, fc1_b_row, sc_wt, sc_b_row):
    return pl.pallas_call(
        _final_body,
        out_shape=(
            jax.ShapeDtypeStruct((1, 1), jnp.float32),
            jax.ShapeDtypeStruct((F3, 1), jnp.float32),
            jax.ShapeDtypeStruct((F3, 1), jnp.float32),
        ),
    )(acc3, deg, b3_row, att_W, tn_wcols, tn_wbt, tn_b_row,
      fc1_wt, fc1_b_row, sc_wt, sc_b_row)


def kernel(features_1, edge_index_1, features_2, edge_index_2,
           W1, b1, W2, b2, W3, b3, att_W, tn_W, tn_Wb, tn_bias,
           fc1_W, fc1_b, sc_W, sc_b):
    f32 = jnp.float32
    src1, dst1 = edge_index_1[0], edge_index_1[1]
    src2, dst2 = edge_index_2[0], edge_index_2[1]

    padn = E_PAD - E
    padidx = (N + (jnp.arange(padn, dtype=jnp.int32) % 16)).astype(jnp.int32)

    def pad_edges(a):
        return jnp.concatenate([a.astype(jnp.int32), padidx])

    src_glob = jnp.concatenate(
        [pad_edges(src1), pad_edges(src2) + NPAD]).reshape(2 * NTILES * NCHUNK, CHUNK)
    dst_all = jnp.concatenate(
        [pad_edges(dst1), pad_edges(dst2)]).reshape(2 * NTILES * NCHUNK, CHUNK)

    zeros_w = jnp.zeros((NPAD, FW), f32)
    ones_chunk = jnp.ones((CHUNK, FW), f32)
    deg = _make_degree()(zeros_w, ones_chunk, dst_all)

    zrows = jnp.zeros((NPAD - N, D), f32)
    xcat = jnp.concatenate([features_1, zrows, features_2, zrows])

    W2p = jnp.zeros((F1, FW), f32).at[:, :F2].set(W2)
    W3p = jnp.zeros((FW, FW), f32).at[:F2, :F3].set(W3)
    b2p = jnp.zeros((FW,), f32).at[:F2].set(b2)

    agg = _make_agg()
    g1 = _dense_first(xcat, deg, W1)
    acc1 = agg(g1, src_glob, dst_all)
    g2 = _dense_mid(acc1, deg, b1.reshape(1, -1), W2p)
    acc2 = agg(g2, src_glob, dst_all)
    g3 = _dense_mid(acc2, deg, b2p.reshape(1, -1), W3p)
    acc3 = agg(g3, src_glob, dst_all)

    tn_wcols = jnp.transpose(tn_W, (1, 2, 0)).reshape(F3, T * F3)
    score, p1, p2 = _final(
        acc3, deg, b3.reshape(1, -1), att_W, tn_wcols,
        tn_Wb.T, tn_bias.reshape(1, -1), fc1_W.T, fc1_b.reshape(1, -1),
        sc_W.T, sc_b.reshape(1, -1))
    return (score, p1, p2)


# speedup vs baseline: 22.5393x; 1.4015x over previous
import functools

import jax
import jax.numpy as jnp
from jax import lax
from jax.experimental import pallas as pl
from jax.experimental.pallas import tpu as pltpu
from jax.experimental.pallas import tpu_sc as plsc

N = 10000
E = 320000
D = 128
F1, F2, F3 = 128, 64, 32
T = 16
BN = 16

NPAD = 10240
NTILES = 16
RPT = NPAD // NTILES
CHUNK = 128
IB = 16
NBLK = 10
NCHUNK = IB * NBLK
EPT = NCHUNK * CHUNK
E_PAD = NTILES * EPT
DEGW = 16


def _sc_mesh():
    return plsc.VectorSubcoreMesh(core_axis_name="c", subcore_axis_name="s",
                                  num_cores=2, num_subcores=NTILES)


@functools.lru_cache(maxsize=None)
def _make_agg(F):
    params = (None if F == 128
              else pltpu.CompilerParams(use_tc_tiling_on_sc=False))

    @functools.partial(
        pl.kernel,
        out_type=jax.ShapeDtypeStruct((2 * NPAD, F), jnp.float32),
        mesh=_sc_mesh(),
        compiler_params=params,
        scratch_types=[
            pltpu.VMEM_SHARED((NPAD, F), jnp.float32),
            pltpu.VMEM((IB, CHUNK), jnp.int32),
            pltpu.VMEM((IB, CHUNK), jnp.int32),
            pltpu.VMEM((CHUNK, F), jnp.float32),
            pltpu.VMEM((CHUNK, F), jnp.float32),
            pltpu.SemaphoreType.DMA,
            pltpu.SemaphoreType.DMA,
            pltpu.SemaphoreType.DMA,
            pltpu.SemaphoreType.DMA,
        ],
    )
    def agg(g_hbm, src_hbm, dst_hbm, out_hbm, acc_sh, sidx, didx,
            rows0, rows1, gsem0, gsem1, ssem0, ssem1):
        c = lax.axis_index("c")
        s = lax.axis_index("s")
        w = c * NTILES + s
        base = c * NPAD + s * RPT
        pltpu.sync_copy(g_hbm.at[pl.ds(base, RPT)], acc_sh.at[pl.ds(s * RPT, RPT)])
        plsc.subcore_barrier()

        def drain_scatter(sem, buf):
            pltpu.make_async_copy(buf, acc_sh.at[pl.ds(0, CHUNK)], sem).wait()

        def drain_gather(sem, buf):
            pltpu.make_async_copy(g_hbm.at[pl.ds(0, CHUNK)], buf, sem).wait()

        def blk(bi, carry):
            @pl.when(bi > 0)
            def _():
                drain_scatter(ssem0, rows0)
                drain_scatter(ssem1, rows1)

            ib = w * NCHUNK + bi * IB
            pltpu.sync_copy(src_hbm.at[pl.ds(ib, IB)], sidx)
            pltpu.sync_copy(dst_hbm.at[pl.ds(ib, IB)], didx)

            def pair(m, carry2):
                ka = 2 * m
                kb = 2 * m + 1

                @pl.when(m > 0)
                def _():
                    drain_scatter(ssem0, rows0)

                pltpu.async_copy(g_hbm.at[sidx.at[ka]], rows0, gsem0)

                @pl.when(m > 0)
                def _():
                    drain_scatter(ssem1, rows1)

                pltpu.async_copy(g_hbm.at[sidx.at[kb]], rows1, gsem1)
                drain_gather(gsem0, rows0)
                pltpu.async_copy(rows0, acc_sh.at[didx.at[ka]], ssem0, add=True)
                drain_gather(gsem1, rows1)
                pltpu.async_copy(rows1, acc_sh.at[didx.at[kb]], ssem1, add=True)
                return carry2

            lax.fori_loop(0, IB // 2, pair, 0)
            return carry

        lax.fori_loop(0, NBLK, blk, 0)
        drain_scatter(ssem0, rows0)
        drain_scatter(ssem1, rows1)
        plsc.subcore_barrier()
        pltpu.sync_copy(acc_sh.at[pl.ds(s * RPT, RPT)], out_hbm.at[pl.ds(base, RPT)])

    return agg


@functools.lru_cache(maxsize=None)
def _make_degree():
    @functools.partial(
        pl.kernel,
        out_type=jax.ShapeDtypeStruct((2 * NPAD, DEGW), jnp.float32),
        mesh=_sc_mesh(),
        compiler_params=pltpu.CompilerParams(use_tc_tiling_on_sc=False),
        scratch_types=[
            pltpu.VMEM_SHARED((NPAD, DEGW), jnp.float32),
            pltpu.VMEM((IB, CHUNK), jnp.int32),
            pltpu.VMEM((CHUNK, DEGW), jnp.float32),
            pltpu.SemaphoreType.DMA,
        ],
    )
    def degree(zeros_hbm, ones_hbm, dst_hbm, out_hbm, acc_sh, didx, ones_v, sem):
        c = lax.axis_index("c")
        s = lax.axis_index("s")
        w = c * NTILES + s
        base = c * NPAD + s * RPT
        pltpu.sync_copy(zeros_hbm.at[pl.ds(s * RPT, RPT)],
                        acc_sh.at[pl.ds(s * RPT, RPT)])
        pltpu.sync_copy(ones_hbm, ones_v)
        plsc.subcore_barrier()

        def blk(bi, carry):
            pltpu.sync_copy(dst_hbm.at[pl.ds(w * NCHUNK + bi * IB, IB)], didx)

            def body(k, carry2):
                pltpu.sync_copy(ones_v, acc_sh.at[didx.at[k]], add=True)
                return carry2

            lax.fori_loop(0, IB, body, 0)
            return carry

        lax.fori_loop(0, NBLK, blk, 0)
        plsc.subcore_barrier()
        pltpu.sync_copy(acc_sh.at[pl.ds(s * RPT, RPT)], out_hbm.at[pl.ds(base, RPT)])

    return degree


_BM = 1024


def _dense_first(xcat, deg, W):
    M, K = xcat.shape
    F = W.shape[1]

    def body(x_ref, deg_ref, w_ref, o_ref):
        dinv = lax.rsqrt(deg_ref[...][:, 0:1] + 1.0)
        o_ref[...] = dinv * jnp.dot(x_ref[...], w_ref[...],
                                    preferred_element_type=jnp.float32)

    return pl.pallas_call(
        body,
        grid=(M // _BM,),
        in_specs=[
            pl.BlockSpec((_BM, K), lambda i: (i, 0)),
            pl.BlockSpec((_BM, DEGW), lambda i: (i, 0)),
            pl.BlockSpec((K, F), lambda i: (0, 0)),
        ],
        out_specs=pl.BlockSpec((_BM, F), lambda i: (i, 0)),
        out_shape=jax.ShapeDtypeStruct((M, F), jnp.float32),
    )(xcat, deg, W)


def _dense_mid(acc, deg, b_row, W):
    M, K = acc.shape
    F = W.shape[1]

    def body(a_ref, deg_ref, b_ref, w_ref, o_ref):
        dinv = lax.rsqrt(deg_ref[...][:, 0:1] + 1.0)
        h = jnp.maximum(dinv * a_ref[...] + b_ref[...], 0.0)
        o_ref[...] = dinv * jnp.dot(h, w_ref[...],
                                    preferred_element_type=jnp.float32)

    return pl.pallas_call(
        body,
        grid=(M // _BM,),
        in_specs=[
            pl.BlockSpec((_BM, K), lambda i: (i, 0)),
            pl.BlockSpec((_BM, DEGW), lambda i: (i, 0)),
            pl.BlockSpec((1, K), lambda i: (0, 0)),
            pl.BlockSpec((K, F), lambda i: (0, 0)),
        ],
        out_specs=pl.BlockSpec((_BM, F), lambda i: (i, 0)),
        out_shape=jax.ShapeDtypeStruct((M, F), jnp.float32),
    )(acc, deg, b_row, W)


def _final_body(acc_ref, deg_ref, b3_ref, attw_ref, tnw_ref, tnwbt_ref,
                tnb_ref, fc1wt_ref, fc1b_ref, scwt_ref, scb_ref,
                score_ref, p1_ref, p2_ref):
    b3 = b3_ref[...]
    attw = attw_ref[...]

    def pooled_rows(lo):
        a = acc_ref[pl.ds(lo, N), :]
        d = deg_ref[pl.ds(lo, N), 0:1]
        dinv = lax.rsqrt(d + 1.0)
        emb = dinv * a + b3
        mean = jnp.mean(emb, axis=0, keepdims=True)
        ctx = jnp.tanh(jnp.dot(mean, attw,
                               preferred_element_type=jnp.float32))
        logits = lax.dot_general(emb, ctx, (((1,), (1,)), ((), ())),
                                 preferred_element_type=jnp.float32)
        sig = jax.nn.sigmoid(logits)
        pooled_col = lax.dot_general(emb, sig, (((0,), (0,)), ((), ())),
                                     preferred_element_type=jnp.float32)
        pooled_row = lax.dot_general(sig, emb, (((0,), (0,)), ((), ())),
                                     preferred_element_type=jnp.float32)
        return pooled_col, pooled_row

    p1c, e1r = pooled_rows(0)
    p2c, e2r = pooled_rows(NPAD)
    p1_ref[...] = p1c
    p2_ref[...] = p2c

    y = jnp.dot(e2r, tnw_ref[...], preferred_element_type=jnp.float32)
    e1_tiled = jnp.concatenate([e1r] * T, axis=1)
    z = y * e1_tiled
    rr = lax.broadcasted_iota(jnp.int32, (T * F3, T), 0)
    cc = lax.broadcasted_iota(jnp.int32, (T * F3, T), 1)
    sel = (rr // F3 == cc).astype(jnp.float32)
    scoring = jnp.dot(z, sel, preferred_element_type=jnp.float32)

    comb = jnp.concatenate([e1r, e2r], axis=1)
    block = jnp.dot(comb, tnwbt_ref[...], preferred_element_type=jnp.float32)
    s = jnp.maximum(scoring + block + tnb_ref[...], 0.0)
    s2 = jnp.maximum(jnp.dot(s, fc1wt_ref[...],
                             preferred_element_type=jnp.float32) + fc1b_ref[...], 0.0)
    score_ref[...] = jax.nn.sigmoid(
        jnp.dot(s2, scwt_ref[...], preferred_element_type=jnp.float32) + scb_ref[...])


def _final(acc3, deg, b3_row, att_W, tn_wcols, tn_wbt, tn_b_row,
           fc1_wt, fc1_b_row, sc_wt, sc_b_row):
    return pl.pallas_call(
        _final_body,
        out_shape=(
            jax.ShapeDtypeStruct((1, 1), jnp.float32),
            jax.ShapeDtypeStruct((F3, 1), jnp.float32),
            jax.ShapeDtypeStruct((F3, 1), jnp.float32),
        ),
    )(acc3, deg, b3_row, att_W, tn_wcols, tn_wbt, tn_b_row,
      fc1_wt, fc1_b_row, sc_wt, sc_b_row)


def kernel(features_1, edge_index_1, features_2, edge_index_2,
           W1, b1, W2, b2, W3, b3, att_W, tn_W, tn_Wb, tn_bias,
           fc1_W, fc1_b, sc_W, sc_b):
    f32 = jnp.float32
    src1, dst1 = edge_index_1[0], edge_index_1[1]
    src2, dst2 = edge_index_2[0], edge_index_2[1]

    padn = E_PAD - E
    padidx = (N + (jnp.arange(padn, dtype=jnp.int32) % 16)).astype(jnp.int32)

    def pad_edges(a):
        return jnp.concatenate([a.astype(jnp.int32), padidx])

    src_glob = jnp.concatenate(
        [pad_edges(src1), pad_edges(src2) + NPAD]).reshape(2 * NTILES * NCHUNK, CHUNK)
    dst_all = jnp.concatenate(
        [pad_edges(dst1), pad_edges(dst2)]).reshape(2 * NTILES * NCHUNK, CHUNK)

    zeros_w = jnp.zeros((NPAD, DEGW), f32)
    ones_chunk = jnp.ones((CHUNK, DEGW), f32)
    deg = _make_degree()(zeros_w, ones_chunk, dst_all)

    zrows = jnp.zeros((NPAD - N, D), f32)
    xcat = jnp.concatenate([features_1, zrows, features_2, zrows])

    g1 = _dense_first(xcat, deg, W1)
    acc1 = _make_agg(F1)(g1, src_glob, dst_all)
    g2 = _dense_mid(acc1, deg, b1.reshape(1, -1), W2)
    acc2 = _make_agg(F2)(g2, src_glob, dst_all)
    g3 = _dense_mid(acc2, deg, b2.reshape(1, -1), W3)
    acc3 = _make_agg(F3)(g3, src_glob, dst_all)

    tn_wcols = jnp.transpose(tn_W, (1, 2, 0)).reshape(F3, T * F3)
    score, p1, p2 = _final(
        acc3, deg, b3.reshape(1, -1), att_W, tn_wcols,
        tn_Wb.T, tn_bias.reshape(1, -1), fc1_W.T, fc1_b.reshape(1, -1),
        sc_W.T, sc_b.reshape(1, -1))
    return (score, p1, p2)


# speedup vs baseline: 26.8546x; 1.1915x over previous
import functools

import jax
import jax.numpy as jnp
from jax import lax
from jax.experimental import pallas as pl
from jax.experimental.pallas import tpu as pltpu
from jax.experimental.pallas import tpu_sc as plsc

N = 10000
E = 320000
D = 128
F1, F2, F3 = 128, 64, 32
T = 16
BN = 16

NPAD = 10240
NTILES = 16
RPT = NPAD // NTILES
CHUNK = 128
IB = 16
NBLK = 10
NCHUNK = IB * NBLK
EPT = NCHUNK * CHUNK
E_PAD = NTILES * EPT
DEGW = 16


def _sc_mesh():
    return plsc.VectorSubcoreMesh(core_axis_name="c", subcore_axis_name="s",
                                  num_cores=2, num_subcores=NTILES)


NBUF = 4


@functools.lru_cache(maxsize=None)
def _make_agg(F):
    params = (None if F == 128
              else pltpu.CompilerParams(use_tc_tiling_on_sc=False))
    chunk = 64 if F == 128 else 128
    ibf = 2048 // chunk
    nchunk = EPT // chunk
    nblk = nchunk // ibf
    ngrp = ibf // NBUF

    @functools.partial(
        pl.kernel,
        out_type=jax.ShapeDtypeStruct((2 * NPAD, F), jnp.float32),
        mesh=_sc_mesh(),
        compiler_params=params,
        scratch_types=[
            pltpu.VMEM_SHARED((NPAD, F), jnp.float32),
            pltpu.VMEM((ibf, chunk), jnp.int32),
            pltpu.VMEM((ibf, chunk), jnp.int32),
        ] + [pltpu.VMEM((chunk, F), jnp.float32)] * NBUF
          + [pltpu.SemaphoreType.DMA] * (2 * NBUF),
    )
    def agg(g_hbm, src_hbm, dst_hbm, out_hbm, acc_sh, sidx, didx, *bufs_sems):
        rows = bufs_sems[:NBUF]
        gsems = bufs_sems[NBUF:2 * NBUF]
        ssems = bufs_sems[2 * NBUF:3 * NBUF]
        c = lax.axis_index("c")
        s = lax.axis_index("s")
        w = c * NTILES + s
        base = c * NPAD + s * RPT
        pltpu.sync_copy(g_hbm.at[pl.ds(base, RPT)], acc_sh.at[pl.ds(s * RPT, RPT)])
        plsc.subcore_barrier()

        def drain_scatter(sem, buf):
            pltpu.make_async_copy(buf, acc_sh.at[pl.ds(0, chunk)], sem).wait()

        def drain_gather(sem, buf):
            pltpu.make_async_copy(g_hbm.at[pl.ds(0, chunk)], buf, sem).wait()

        def blk(bi, carry):
            @pl.when(bi > 0)
            def _():
                for t in range(NBUF):
                    drain_scatter(ssems[t], rows[t])

            ib = w * nchunk + bi * ibf
            pltpu.sync_copy(src_hbm.at[pl.ds(ib, ibf)], sidx)
            pltpu.sync_copy(dst_hbm.at[pl.ds(ib, ibf)], didx)

            def grp(q, carry2):
                for t in range(NBUF):
                    @pl.when(q > 0)
                    def _(t=t):
                        drain_scatter(ssems[t], rows[t])

                    pltpu.async_copy(g_hbm.at[sidx.at[NBUF * q + t]],
                                     rows[t], gsems[t])
                for t in range(NBUF):
                    drain_gather(gsems[t], rows[t])
                    pltpu.async_copy(rows[t], acc_sh.at[didx.at[NBUF * q + t]],
                                     ssems[t], add=True)
                return carry2

            lax.fori_loop(0, ngrp, grp, 0)
            return carry

        lax.fori_loop(0, nblk, blk, 0)
        for t in range(NBUF):
            drain_scatter(ssems[t], rows[t])
        plsc.subcore_barrier()
        pltpu.sync_copy(acc_sh.at[pl.ds(s * RPT, RPT)], out_hbm.at[pl.ds(base, RPT)])

    return agg


@functools.lru_cache(maxsize=None)
def _make_degree():
    @functools.partial(
        pl.kernel,
        out_type=jax.ShapeDtypeStruct((2 * NPAD, DEGW), jnp.float32),
        mesh=_sc_mesh(),
        compiler_params=pltpu.CompilerParams(use_tc_tiling_on_sc=False),
        scratch_types=[
            pltpu.VMEM_SHARED((NPAD, DEGW), jnp.float32),
            pltpu.VMEM((IB, CHUNK), jnp.int32),
            pltpu.VMEM((CHUNK, DEGW), jnp.float32),
            pltpu.SemaphoreType.DMA,
        ],
    )
    def degree(zeros_hbm, ones_hbm, dst_hbm, out_hbm, acc_sh, didx, ones_v, sem):
        c = lax.axis_index("c")
        s = lax.axis_index("s")
        w = c * NTILES + s
        base = c * NPAD + s * RPT
        pltpu.sync_copy(zeros_hbm.at[pl.ds(s * RPT, RPT)],
                        acc_sh.at[pl.ds(s * RPT, RPT)])
        pltpu.sync_copy(ones_hbm, ones_v)
        plsc.subcore_barrier()

        def blk(bi, carry):
            pltpu.sync_copy(dst_hbm.at[pl.ds(w * NCHUNK + bi * IB, IB)], didx)

            def body(k, carry2):
                pltpu.sync_copy(ones_v, acc_sh.at[didx.at[k]], add=True)
                return carry2

            lax.fori_loop(0, IB, body, 0)
            return carry

        lax.fori_loop(0, NBLK, blk, 0)
        plsc.subcore_barrier()
        pltpu.sync_copy(acc_sh.at[pl.ds(s * RPT, RPT)], out_hbm.at[pl.ds(base, RPT)])

    return degree


_BM = 1024


def _dense_first(xcat, deg, W):
    M, K = xcat.shape
    F = W.shape[1]

    def body(x_ref, deg_ref, w_ref, o_ref):
        dinv = lax.rsqrt(deg_ref[...][:, 0:1] + 1.0)
        o_ref[...] = dinv * jnp.dot(x_ref[...], w_ref[...],
                                    preferred_element_type=jnp.float32)

    return pl.pallas_call(
        body,
        grid=(M // _BM,),
        in_specs=[
            pl.BlockSpec((_BM, K), lambda i: (i, 0)),
            pl.BlockSpec((_BM, DEGW), lambda i: (i, 0)),
            pl.BlockSpec((K, F), lambda i: (0, 0)),
        ],
        out_specs=pl.BlockSpec((_BM, F), lambda i: (i, 0)),
        out_shape=jax.ShapeDtypeStruct((M, F), jnp.float32),
    )(xcat, deg, W)


def _dense_mid(acc, deg, b_row, W):
    M, K = acc.shape
    F = W.shape[1]

    def body(a_ref, deg_ref, b_ref, w_ref, o_ref):
        dinv = lax.rsqrt(deg_ref[...][:, 0:1] + 1.0)
        h = jnp.maximum(dinv * a_ref[...] + b_ref[...], 0.0)
        o_ref[...] = dinv * jnp.dot(h, w_ref[...],
                                    preferred_element_type=jnp.float32)

    return pl.pallas_call(
        body,
        grid=(M // _BM,),
        in_specs=[
            pl.BlockSpec((_BM, K), lambda i: (i, 0)),
            pl.BlockSpec((_BM, DEGW), lambda i: (i, 0)),
            pl.BlockSpec((1, K), lambda i: (0, 0)),
            pl.BlockSpec((K, F), lambda i: (0, 0)),
        ],
        out_specs=pl.BlockSpec((_BM, F), lambda i: (i, 0)),
        out_shape=jax.ShapeDtypeStruct((M, F), jnp.float32),
    )(acc, deg, b_row, W)


def _final_body(acc_ref, deg_ref, b3_ref, attw_ref, tnw_ref, tnwbt_ref,
                tnb_ref, fc1wt_ref, fc1b_ref, scwt_ref, scb_ref,
                score_ref, p1_ref, p2_ref):
    b3 = b3_ref[...]
    attw = attw_ref[...]

    def pooled_rows(lo):
        a = acc_ref[pl.ds(lo, N), :]
        d = deg_ref[pl.ds(lo, N), 0:1]
        dinv = lax.rsqrt(d + 1.0)
        emb = dinv * a + b3
        mean = jnp.mean(emb, axis=0, keepdims=True)
        ctx = jnp.tanh(jnp.dot(mean, attw,
                               preferred_element_type=jnp.float32))
        logits = lax.dot_general(emb, ctx, (((1,), (1,)), ((), ())),
                                 preferred_element_type=jnp.float32)
        sig = jax.nn.sigmoid(logits)
        pooled_col = lax.dot_general(emb, sig, (((0,), (0,)), ((), ())),
                                     preferred_element_type=jnp.float32)
        pooled_row = lax.dot_general(sig, emb, (((0,), (0,)), ((), ())),
                                     preferred_element_type=jnp.float32)
        return pooled_col, pooled_row

    p1c, e1r = pooled_rows(0)
    p2c, e2r = pooled_rows(NPAD)
    p1_ref[...] = p1c
    p2_ref[...] = p2c

    y = jnp.dot(e2r, tnw_ref[...], preferred_element_type=jnp.float32)
    e1_tiled = jnp.concatenate([e1r] * T, axis=1)
    z = y * e1_tiled
    rr = lax.broadcasted_iota(jnp.int32, (T * F3, T), 0)
    cc = lax.broadcasted_iota(jnp.int32, (T * F3, T), 1)
    sel = (rr // F3 == cc).astype(jnp.float32)
    scoring = jnp.dot(z, sel, preferred_element_type=jnp.float32)

    comb = jnp.concatenate([e1r, e2r], axis=1)
    block = jnp.dot(comb, tnwbt_ref[...], preferred_element_type=jnp.float32)
    s = jnp.maximum(scoring + block + tnb_ref[...], 0.0)
    s2 = jnp.maximum(jnp.dot(s, fc1wt_ref[...],
                             preferred_element_type=jnp.float32) + fc1b_ref[...], 0.0)
    score_ref[...] = jax.nn.sigmoid(
        jnp.dot(s2, scwt_ref[...], preferred_element_type=jnp.float32) + scb_ref[...])


def _final(acc3, deg, b3_row, att_W, tn_wcols, tn_wbt, tn_b_row,
           fc1_wt, fc1_b_row, sc_wt, sc_b_row):
    return pl.pallas_call(
        _final_body,
        out_shape=(
            jax.ShapeDtypeStruct((1, 1), jnp.float32),
            jax.ShapeDtypeStruct((F3, 1), jnp.float32),
            jax.ShapeDtypeStruct((F3, 1), jnp.float32),
        ),
    )(acc3, deg, b3_row, att_W, tn_wcols, tn_wbt, tn_b_row,
      fc1_wt, fc1_b_row, sc_wt, sc_b_row)


def kernel(features_1, edge_index_1, features_2, edge_index_2,
           W1, b1, W2, b2, W3, b3, att_W, tn_W, tn_Wb, tn_bias,
           fc1_W, fc1_b, sc_W, sc_b):
    f32 = jnp.float32
    src1, dst1 = edge_index_1[0], edge_index_1[1]
    src2, dst2 = edge_index_2[0], edge_index_2[1]

    padn = E_PAD - E
    padidx = (N + (jnp.arange(padn, dtype=jnp.int32) % 16)).astype(jnp.int32)

    def pad_edges(a):
        return jnp.concatenate([a.astype(jnp.int32), padidx])

    src_flat = jnp.concatenate([pad_edges(src1), pad_edges(src2) + NPAD])
    dst_flat = jnp.concatenate([pad_edges(dst1), pad_edges(dst2)])
    dst_all = dst_flat.reshape(2 * NTILES * NCHUNK, CHUNK)

    zeros_w = jnp.zeros((NPAD, DEGW), f32)
    ones_chunk = jnp.ones((CHUNK, DEGW), f32)
    deg = _make_degree()(zeros_w, ones_chunk, dst_all)

    zrows = jnp.zeros((NPAD - N, D), f32)
    xcat = jnp.concatenate([features_1, zrows, features_2, zrows])

    def run_agg(F, g):
        ch = 64 if F == 128 else 128
        return _make_agg(F)(g, src_flat.reshape(-1, ch), dst_flat.reshape(-1, ch))

    g1 = _dense_first(xcat, deg, W1)
    acc1 = run_agg(F1, g1)
    g2 = _dense_mid(acc1, deg, b1.reshape(1, -1), W2)
    acc2 = run_agg(F2, g2)
    g3 = _dense_mid(acc2, deg, b2.reshape(1, -1), W3)
    acc3 = run_agg(F3, g3)

    tn_wcols = jnp.transpose(tn_W, (1, 2, 0)).reshape(F3, T * F3)
    score, p1, p2 = _final(
        acc3, deg, b3.reshape(1, -1), att_W, tn_wcols,
        tn_Wb.T, tn_bias.reshape(1, -1), fc1_W.T, fc1_b.reshape(1, -1),
        sc_W.T, sc_b.reshape(1, -1))
    return (score, p1, p2)


# speedup vs baseline: 28.2095x; 1.0505x over previous
import functools

import jax
import jax.numpy as jnp
from jax import lax
from jax.experimental import pallas as pl
from jax.experimental.pallas import tpu as pltpu
from jax.experimental.pallas import tpu_sc as plsc

N = 10000
E = 320000
D = 128
F1, F2, F3 = 128, 64, 32
T = 16
BN = 16

NPAD = 10240
NTILES = 16
RPT = NPAD // NTILES
CHUNK = 128
IB = 16
NBLK = 10
NCHUNK = IB * NBLK
EPT = NCHUNK * CHUNK
E_PAD = NTILES * EPT
DEGW = 16
DEG_IR = 32


def _sc_mesh():
    return plsc.VectorSubcoreMesh(core_axis_name="c", subcore_axis_name="s",
                                  num_cores=2, num_subcores=NTILES)


NBUF = 4


def _agg_config(F):
    if F == 128:
        return 64, 1, 2048
    return 256, 2, 4096


@functools.lru_cache(maxsize=None)
def _make_agg(F):
    params = (None if F == 128
              else pltpu.CompilerParams(use_tc_tiling_on_sc=False))
    chunk, slices, blk_edges = _agg_config(F)
    iw = chunk // slices
    irows = blk_edges // iw
    cpb = blk_edges // chunk
    ngrp = cpb // NBUF
    nblk = EPT // blk_edges

    @functools.partial(
        pl.kernel,
        out_type=jax.ShapeDtypeStruct((2 * NPAD, F), jnp.float32),
        mesh=_sc_mesh(),
        compiler_params=params,
        scratch_types=[
            pltpu.VMEM_SHARED((NPAD, F), jnp.float32),
            pltpu.VMEM((irows, iw), jnp.int32),
            pltpu.VMEM((irows, iw), jnp.int32),
        ] + [pltpu.VMEM((chunk, F), jnp.float32)] * NBUF
          + [pltpu.SemaphoreType.DMA] * (2 * NBUF),
    )
    def agg(g_hbm, src_hbm, dst_hbm, out_hbm, acc_sh, sidx, didx, *bufs_sems):
        rows = bufs_sems[:NBUF]
        gsems = bufs_sems[NBUF:2 * NBUF]
        ssems = bufs_sems[2 * NBUF:3 * NBUF]
        c = lax.axis_index("c")
        s = lax.axis_index("s")
        w = c * NTILES + s
        base = c * NPAD + s * RPT
        pltpu.sync_copy(g_hbm.at[pl.ds(base, RPT)], acc_sh.at[pl.ds(s * RPT, RPT)])
        plsc.subcore_barrier()

        def drain_scatter(sem, buf):
            for u in range(slices):
                pltpu.make_async_copy(buf.at[pl.ds(u * iw, iw)],
                                      acc_sh.at[pl.ds(0, iw)], sem).wait()

        def drain_gather(sem, buf):
            for u in range(slices):
                pltpu.make_async_copy(g_hbm.at[pl.ds(0, iw)],
                                      buf.at[pl.ds(u * iw, iw)], sem).wait()

        def blk(bi, carry):
            @pl.when(bi > 0)
            def _():
                for t in range(NBUF):
                    drain_scatter(ssems[t], rows[t])

            ib = w * (nblk * irows) + bi * irows
            pltpu.sync_copy(src_hbm.at[pl.ds(ib, irows)], sidx)
            pltpu.sync_copy(dst_hbm.at[pl.ds(ib, irows)], didx)

            def grp(q, carry2):
                for t in range(NBUF):
                    @pl.when(q > 0)
                    def _(t=t):
                        drain_scatter(ssems[t], rows[t])

                    kc = NBUF * q + t
                    for u in range(slices):
                        pltpu.async_copy(
                            g_hbm.at[sidx.at[kc * slices + u]],
                            rows[t].at[pl.ds(u * iw, iw)], gsems[t])
                for t in range(NBUF):
                    drain_gather(gsems[t], rows[t])
                    kc = NBUF * q + t
                    for u in range(slices):
                        pltpu.async_copy(
                            rows[t].at[pl.ds(u * iw, iw)],
                            acc_sh.at[didx.at[kc * slices + u]],
                            ssems[t], add=True)
                return carry2

            lax.fori_loop(0, ngrp, grp, 0)
            return carry

        lax.fori_loop(0, nblk, blk, 0)
        for t in range(NBUF):
            drain_scatter(ssems[t], rows[t])
        plsc.subcore_barrier()
        pltpu.sync_copy(acc_sh.at[pl.ds(s * RPT, RPT)], out_hbm.at[pl.ds(base, RPT)])

    return agg


@functools.lru_cache(maxsize=None)
def _make_degree():
    @functools.partial(
        pl.kernel,
        out_type=jax.ShapeDtypeStruct((2 * NPAD, DEGW), jnp.float32),
        mesh=_sc_mesh(),
        compiler_params=pltpu.CompilerParams(use_tc_tiling_on_sc=False),
        scratch_types=[
            pltpu.VMEM_SHARED((NPAD, DEGW), jnp.float32),
            pltpu.VMEM((DEG_IR, CHUNK), jnp.int32),
            pltpu.VMEM((CHUNK, DEGW), jnp.float32),
            pltpu.SemaphoreType.DMA,
        ],
    )
    def degree(zeros_hbm, ones_hbm, dst_hbm, out_hbm, acc_sh, didx, ones_v, sem):
        c = lax.axis_index("c")
        s = lax.axis_index("s")
        w = c * NTILES + s
        base = c * NPAD + s * RPT
        nblk = EPT // (DEG_IR * CHUNK)
        grps = DEG_IR // 8
        pltpu.sync_copy(zeros_hbm.at[pl.ds(s * RPT, RPT)],
                        acc_sh.at[pl.ds(s * RPT, RPT)])
        pltpu.sync_copy(ones_hbm, ones_v)
        plsc.subcore_barrier()

        def blk(bi, carry):
            pltpu.sync_copy(dst_hbm.at[pl.ds(w * (nblk * DEG_IR) + bi * DEG_IR,
                                             DEG_IR)], didx)
            def grp(q, carry2):
                for k in range(8):
                    pltpu.async_copy(ones_v, acc_sh.at[didx.at[8 * q + k]],
                                     sem, add=True)
                for k in range(8):
                    pltpu.make_async_copy(ones_v, acc_sh.at[pl.ds(0, CHUNK)],
                                          sem).wait()
                return carry2

            lax.fori_loop(0, grps, grp, 0)
            return carry

        lax.fori_loop(0, nblk, blk, 0)
        plsc.subcore_barrier()
        pltpu.sync_copy(acc_sh.at[pl.ds(s * RPT, RPT)], out_hbm.at[pl.ds(base, RPT)])

    return degree


_BM = 1024


def _dense_first(xcat, deg, W):
    M, K = xcat.shape
    F = W.shape[1]

    def body(x_ref, deg_ref, w_ref, o_ref):
        dinv = lax.rsqrt(deg_ref[...][:, 0:1] + 1.0)
        o_ref[...] = dinv * jnp.dot(x_ref[...], w_ref[...],
                                    preferred_element_type=jnp.float32)

    return pl.pallas_call(
        body,
        grid=(M // _BM,),
        in_specs=[
            pl.BlockSpec((_BM, K), lambda i: (i, 0)),
            pl.BlockSpec((_BM, DEGW), lambda i: (i, 0)),
            pl.BlockSpec((K, F), lambda i: (0, 0)),
        ],
        out_specs=pl.BlockSpec((_BM, F), lambda i: (i, 0)),
        out_shape=jax.ShapeDtypeStruct((M, F), jnp.float32),
    )(xcat, deg, W)


def _dense_mid(acc, deg, b_row, W):
    M, K = acc.shape
    F = W.shape[1]

    def body(a_ref, deg_ref, b_ref, w_ref, o_ref):
        dinv = lax.rsqrt(deg_ref[...][:, 0:1] + 1.0)
        h = jnp.maximum(dinv * a_ref[...] + b_ref[...], 0.0)
        o_ref[...] = dinv * jnp.dot(h, w_ref[...],
                                    preferred_element_type=jnp.float32)

    return pl.pallas_call(
        body,
        grid=(M // _BM,),
        in_specs=[
            pl.BlockSpec((_BM, K), lambda i: (i, 0)),
            pl.BlockSpec((_BM, DEGW), lambda i: (i, 0)),
            pl.BlockSpec((1, K), lambda i: (0, 0)),
            pl.BlockSpec((K, F), lambda i: (0, 0)),
        ],
        out_specs=pl.BlockSpec((_BM, F), lambda i: (i, 0)),
        out_shape=jax.ShapeDtypeStruct((M, F), jnp.float32),
    )(acc, deg, b_row, W)


def _final_body(acc_ref, deg_ref, b3_ref, attw_ref, tnw_ref, tnwbt_ref,
                tnb_ref, fc1wt_ref, fc1b_ref, scwt_ref, scb_ref,
                score_ref, p1_ref, p2_ref):
    b3 = b3_ref[...]
    attw = attw_ref[...]

    def pooled_rows(lo):
        a = acc_ref[pl.ds(lo, N), :]
        d = deg_ref[pl.ds(lo, N), 0:1]
        dinv = lax.rsqrt(d + 1.0)
        emb = dinv * a + b3
        mean = jnp.mean(emb, axis=0, keepdims=True)
        ctx = jnp.tanh(jnp.dot(mean, attw,
                               preferred_element_type=jnp.float32))
        logits = lax.dot_general(emb, ctx, (((1,), (1,)), ((), ())),
                                 preferred_element_type=jnp.float32)
        sig = jax.nn.sigmoid(logits)
        pooled_col = lax.dot_general(emb, sig, (((0,), (0,)), ((), ())),
                                     preferred_element_type=jnp.float32)
        pooled_row = lax.dot_general(sig, emb, (((0,), (0,)), ((), ())),
                                     preferred_element_type=jnp.float32)
        return pooled_col, pooled_row

    p1c, e1r = pooled_rows(0)
    p2c, e2r = pooled_rows(NPAD)
    p1_ref[...] = p1c
    p2_ref[...] = p2c

    y = jnp.dot(e2r, tnw_ref[...], preferred_element_type=jnp.float32)
    e1_tiled = jnp.concatenate([e1r] * T, axis=1)
    z = y * e1_tiled
    rr = lax.broadcasted_iota(jnp.int32, (T * F3, T), 0)
    cc = lax.broadcasted_iota(jnp.int32, (T * F3, T), 1)
    sel = (rr // F3 == cc).astype(jnp.float32)
    scoring = jnp.dot(z, sel, preferred_element_type=jnp.float32)

    comb = jnp.concatenate([e1r, e2r], axis=1)
    block = jnp.dot(comb, tnwbt_ref[...], preferred_element_type=jnp.float32)
    s = jnp.maximum(scoring + block + tnb_ref[...], 0.0)
    s2 = jnp.maximum(jnp.dot(s, fc1wt_ref[...],
                             preferred_element_type=jnp.float32) + fc1b_ref[...], 0.0)
    score_ref[...] = jax.nn.sigmoid(
        jnp.dot(s2, scwt_ref[...], preferred_element_type=jnp.float32) + scb_ref[...])


def _final(acc3, deg, b3_row, att_W, tn_wcols, tn_wbt, tn_b_row,
           fc1_wt, fc1_b_row, sc_wt, sc_b_row):
    return pl.pallas_call(
        _final_body,
        out_shape=(
            jax.ShapeDtypeStruct((1, 1), jnp.float32),
            jax.ShapeDtypeStruct((F3, 1), jnp.float32),
            jax.ShapeDtypeStruct((F3, 1), jnp.float32),
        ),
    )(acc3, deg, b3_row, att_W, tn_wcols, tn_wbt, tn_b_row,
      fc1_wt, fc1_b_row, sc_wt, sc_b_row)


def kernel(features_1, edge_index_1, features_2, edge_index_2,
           W1, b1, W2, b2, W3, b3, att_W, tn_W, tn_Wb, tn_bias,
           fc1_W, fc1_b, sc_W, sc_b):
    f32 = jnp.float32
    src1, dst1 = edge_index_1[0], edge_index_1[1]
    src2, dst2 = edge_index_2[0], edge_index_2[1]

    padn = E_PAD - E
    padidx = (N + (jnp.arange(padn, dtype=jnp.int32) % 16)).astype(jnp.int32)

    def pad_edges(a):
        return jnp.concatenate([a.astype(jnp.int32), padidx])

    src_flat = jnp.concatenate([pad_edges(src1), pad_edges(src2) + NPAD])
    dst_flat = jnp.concatenate([pad_edges(dst1), pad_edges(dst2)])
    dst_all = dst_flat.reshape(2 * NTILES * NCHUNK, CHUNK)

    zeros_w = jnp.zeros((NPAD, DEGW), f32)
    ones_chunk = jnp.ones((CHUNK, DEGW), f32)
    deg = _make_degree()(zeros_w, ones_chunk, dst_all)

    zrows = jnp.zeros((NPAD - N, D), f32)
    xcat = jnp.concatenate([features_1, zrows, features_2, zrows])

    def run_agg(F, g):
        chunk, slices, _ = _agg_config(F)
        iw = chunk // slices
        return _make_agg(F)(g, src_flat.reshape(-1, iw), dst_flat.reshape(-1, iw))

    g1 = _dense_first(xcat, deg, W1)
    acc1 = run_agg(F1, g1)
    g2 = _dense_mid(acc1, deg, b1.reshape(1, -1), W2)
    acc2 = run_agg(F2, g2)
    g3 = _dense_mid(acc2, deg, b2.reshape(1, -1), W3)
    acc3 = run_agg(F3, g3)

    tn_wcols = jnp.transpose(tn_W, (1, 2, 0)).reshape(F3, T * F3)
    score, p1, p2 = _final(
        acc3, deg, b3.reshape(1, -1), att_W, tn_wcols,
        tn_Wb.T, tn_bias.reshape(1, -1), fc1_W.T, fc1_b.reshape(1, -1),
        sc_W.T, sc_b.reshape(1, -1))
    return (score, p1, p2)
